# Initial kernel scaffold; baseline (speedup 1.0000x reference)
import jax
import jax.numpy as jnp
from jax.experimental import pallas as pl


def kernel(x):
    raise NotImplementedError("write your pallas kernel here")


# speedup vs baseline: 2.1226x; 2.1226x over previous
import jax
import jax.numpy as jnp
from jax import lax
from jax.experimental import pallas as pl

K = 64
ROWS_PER_BLOCK = 128
NEG_INF = float("-inf")


def _topk_block(x_ref, o_ref):
    x = x_ref[0]
    R, N = x.shape
    iota = lax.broadcasted_iota(jnp.int32, (R, N), 1)
    oiota = lax.broadcasted_iota(jnp.int32, (R, K), 1)

    def body(k, carry):
        x, acc = carry
        m = jnp.max(x, axis=1, keepdims=True)
        idx = jnp.min(jnp.where(x == m, iota, jnp.int32(N)), axis=1, keepdims=True)
        x = jnp.where(iota == idx, NEG_INF, x)
        acc = jnp.where(oiota == k, m, acc)
        return x, acc

    _, acc = lax.fori_loop(0, K, body, (x, jnp.full((R, K), NEG_INF, jnp.float32)))
    o_ref[0] = acc


def kernel(x):
    B, S, N = x.shape
    grid = (B, S // ROWS_PER_BLOCK)
    return pl.pallas_call(
        _topk_block,
        grid=grid,
        in_specs=[pl.BlockSpec((1, ROWS_PER_BLOCK, N), lambda b, i: (b, i, 0))],
        out_specs=pl.BlockSpec((1, ROWS_PER_BLOCK, K), lambda b, i: (b, i, 0)),
        out_shape=jax.ShapeDtypeStruct((B, S, K), jnp.float32),
    )(x)


# speedup vs baseline: 24.0480x; 11.3297x over previous
import functools

import jax
import jax.numpy as jnp
from jax import lax
from jax.experimental import pallas as pl
from jax.experimental.pallas import tpu as pltpu
from jax.experimental.pallas import tpu_sc as plsc

K = 64
N = 8192
L = 16
R_TOTAL = 8192
NW = 32
ROWS_PER_W = R_TOTAL // NW
BATCH = 8
GROUP = 1024
NGROUPS = N // GROUP


def _rev(v):
    return lax.rev(v, (0,))


def _sort_bitonic(vs):
    if len(vs) == 1:
        return [jnp.sort(vs[0])]
    h = len(vs) // 2
    lo = [jnp.minimum(a, b) for a, b in zip(vs[:h], vs[h:])]
    hi = [jnp.maximum(a, b) for a, b in zip(vs[:h], vs[h:])]
    return _sort_bitonic(lo) + _sort_bitonic(hi)


def _merge(A, B, cap=False):
    rb = [_rev(b) for b in reversed(B)]
    lo = [jnp.minimum(a, r) for a, r in zip(A, rb)]
    hi = [jnp.maximum(a, r) for a, r in zip(A, rb)]
    if cap:
        return _sort_bitonic(hi)
    return _sort_bitonic(lo) + _sort_bitonic(hi)


def _top64_of_group(load):
    lists = [[jnp.sort(load(i))] for i in range(GROUP // L)]
    while len(lists) > 1:
        nxt = []
        for a, b in zip(lists[0::2], lists[1::2]):
            nxt.append(_merge(a, b, cap=(len(a) == 4)))
        lists = nxt
    return lists[0]


def _sc_topk(x_hbm, out_hbm, row_v, cand_v, out_v, sem):
    wid = lax.axis_index("s") * 2 + lax.axis_index("c")
    base = wid * ROWS_PER_W

    def batch_body(b, _):
        rows0 = base + b * BATCH
        copy = pltpu.make_async_copy(
            x_hbm.at[pl.ds(rows0 * N, BATCH * N)], row_v, sem)
        copy.start()
        copy.wait()

        def row_body(i, _):
            def group_body(g, _):
                off = i * N + g * GROUP

                def load(j):
                    return row_v[pl.ds(off + j * L, L)]

                top = _top64_of_group(load)
                for j in range(4):
                    cand_v[pl.ds(g * K + j * L, L)] = top[j]
                return 0

            lax.fori_loop(0, NGROUPS, group_body, 0, unroll=False)

            lists = []
            for g in range(NGROUPS):
                lists.append([cand_v[pl.ds(g * K + j * L, L)]
                              for j in range(4)])
            while len(lists) > 1:
                lists = [_merge(a, b, cap=True)
                         for a, b in zip(lists[0::2], lists[1::2])]
            top = lists[0]
            r = b * BATCH + i
            for j in range(4):
                out_v[pl.ds(r * K + j * L, L)] = _rev(top[3 - j])
            return 0

        lax.fori_loop(0, BATCH, row_body, 0, unroll=False)
        return 0

    lax.fori_loop(0, ROWS_PER_W // BATCH, batch_body, 0, unroll=False)

    out_copy = pltpu.make_async_copy(
        out_v, out_hbm.at[pl.ds(base * K, ROWS_PER_W * K)], sem)
    out_copy.start()
    out_copy.wait()


@jax.jit
def kernel(x):
    B, S, _ = x.shape
    xf = x.reshape(R_TOTAL, N)
    mesh = plsc.VectorSubcoreMesh(core_axis_name="c", subcore_axis_name="s")
    run = pl.kernel(
        _sc_topk,
        out_type=jax.ShapeDtypeStruct((R_TOTAL * K,), jnp.float32),
        mesh=mesh,
        compiler_params=pltpu.CompilerParams(needs_layout_passes=False),
        scratch_types=[
            pltpu.VMEM((BATCH * N,), jnp.float32),
            pltpu.VMEM((NGROUPS * K,), jnp.float32),
            pltpu.VMEM((ROWS_PER_W * K,), jnp.float32),
            pltpu.SemaphoreType.DMA,
        ],
    )
    out = run(xf.reshape(R_TOTAL * N))
    return out.reshape(B, S, K)


# speedup vs baseline: 32.9656x; 1.3708x over previous
import functools

import jax
import jax.numpy as jnp
from jax import lax
from jax.experimental import pallas as pl
from jax.experimental.pallas import tpu as pltpu
from jax.experimental.pallas import tpu_sc as plsc

K = 64
N = 8192
L = 16
R_TOTAL = 8192
NW = 32
ROWS_PER_W = R_TOTAL // NW
BATCH = 8
NGRP = 32


def _rev(v):
    return lax.rev(v, (0,))


def _sort_bitonic(vs):
    if len(vs) == 1:
        return [jnp.sort(vs[0])]
    h = len(vs) // 2
    lo = [jnp.minimum(a, b) for a, b in zip(vs[:h], vs[h:])]
    hi = [jnp.maximum(a, b) for a, b in zip(vs[:h], vs[h:])]
    return _sort_bitonic(lo) + _sort_bitonic(hi)


def _merge(A, B, cap=False):
    rb = [_rev(b) for b in reversed(B)]
    lo = [jnp.minimum(a, r) for a, r in zip(A, rb)]
    hi = [jnp.maximum(a, r) for a, r in zip(A, rb)]
    if cap:
        return _sort_bitonic(hi)
    return _sort_bitonic(lo) + _sort_bitonic(hi)


def _kv_sort_bitonic(ks, vs):
    if len(ks) == 1:
        sk, sv = plsc.sort_key_val(ks[0], vs[0])
        return [sk], [sv]
    h = len(ks) // 2
    m = [a <= b for a, b in zip(ks[:h], ks[h:])]
    lok = [jnp.where(mm, a, b) for mm, a, b in zip(m, ks[:h], ks[h:])]
    hik = [jnp.where(mm, b, a) for mm, a, b in zip(m, ks[:h], ks[h:])]
    lov = [jnp.where(mm, a, b) for mm, a, b in zip(m, vs[:h], vs[h:])]
    hiv = [jnp.where(mm, b, a) for mm, a, b in zip(m, vs[:h], vs[h:])]
    lk, lv = _kv_sort_bitonic(lok, lov)
    hk, hv = _kv_sort_bitonic(hik, hiv)
    return lk + hk, lv + hv


def _kv_merge(Ak, Av, Bk, Bv, cap=False):
    rbk = [_rev(b) for b in reversed(Bk)]
    rbv = [_rev(b) for b in reversed(Bv)]
    m = [a <= rb for a, rb in zip(Ak, rbk)]
    lok = [jnp.where(mm, a, b) for mm, a, b in zip(m, Ak, rbk)]
    hik = [jnp.where(mm, b, a) for mm, a, b in zip(m, Ak, rbk)]
    lov = [jnp.where(mm, a, b) for mm, a, b in zip(m, Av, rbv)]
    hiv = [jnp.where(mm, b, a) for mm, a, b in zip(m, Av, rbv)]
    if cap:
        return _kv_sort_bitonic(hik, hiv)
    lk, lv = _kv_sort_bitonic(lok, lov)
    hk, hv = _kv_sort_bitonic(hik, hiv)
    return lk + hk, lv + hv


def _process_row(row_v, out_v, off, r):
    iota = lax.iota(jnp.int32, L)
    lists = []
    for g in range(NGRP):
        cm = row_v[pl.ds(off + g * 256, L)]
        for j in range(1, 16):
            cm = jnp.maximum(cm, row_v[pl.ds(off + g * 256 + j * L, L)])
        sk, sv = plsc.sort_key_val(cm, g * 256 + iota)
        lists.append(([sk], [sv]))
    while len(lists) > 1:
        nxt = []
        for (ak, av), (bk, bv) in zip(lists[0::2], lists[1::2]):
            nxt.append(_kv_merge(ak, av, bk, bv, cap=(len(ak) == 4)))
        lists = nxt
    vals4 = lists[0][1]
    leaves = []
    for v in vals4:
        base = v + off
        for j in range(16):
            leaves.append(jnp.sort(plsc.load_gather(row_v, [base + j * L])))
    ls = [[x] for x in leaves]
    while len(ls) > 1:
        ls = [_merge(a, b, cap=(len(a) == 4))
              for a, b in zip(ls[0::2], ls[1::2])]
    top = ls[0]
    for j in range(4):
        out_v[pl.ds(r * K + j * L, L)] = _rev(top[3 - j])


def _sc_topk(x_hbm, out_hbm, row_v, out_v, sem):
    wid = lax.axis_index("s") * 2 + lax.axis_index("c")
    base = wid * ROWS_PER_W

    def batch_body(b, _):
        rows0 = base + b * BATCH
        copy = pltpu.make_async_copy(
            x_hbm.at[pl.ds(rows0 * N, BATCH * N)], row_v, sem)
        copy.start()
        copy.wait()

        def row_body(i, _):
            _process_row(row_v, out_v, i * N, b * BATCH + i)
            return 0

        lax.fori_loop(0, BATCH, row_body, 0, unroll=False)
        return 0

    lax.fori_loop(0, ROWS_PER_W // BATCH, batch_body, 0, unroll=False)

    out_copy = pltpu.make_async_copy(
        out_v, out_hbm.at[pl.ds(base * K, ROWS_PER_W * K)], sem)
    out_copy.start()
    out_copy.wait()


@jax.jit
def kernel(x):
    B, S, _ = x.shape
    mesh = plsc.VectorSubcoreMesh(core_axis_name="c", subcore_axis_name="s")
    run = pl.kernel(
        _sc_topk,
        out_type=jax.ShapeDtypeStruct((R_TOTAL * K,), jnp.float32),
        mesh=mesh,
        compiler_params=pltpu.CompilerParams(needs_layout_passes=False),
        scratch_types=[
            pltpu.VMEM((BATCH * N,), jnp.float32),
            pltpu.VMEM((ROWS_PER_W * K,), jnp.float32),
            pltpu.SemaphoreType.DMA,
        ],
    )
    out = run(x.reshape(R_TOTAL * N))
    return out.reshape(B, S, K)
